# baseline (speedup 1.0000x reference)
import functools

import jax
import jax.numpy as jnp
from jax import lax
from jax.experimental import pallas as pl
from jax.experimental.pallas import tpu as pltpu
from jax.experimental.pallas import tpu_sc as plsc

TOPK = 64
LANES = 16
BINS = 256
HSTRIDE = BINS + 17
UNROLL = 4


def _mono_u32(v):
    u = lax.bitcast_convert_type(v, jnp.uint32)
    msb = jnp.uint32(0x80000000)
    return jnp.where(u >= msb, u ^ jnp.uint32(0xFFFFFFFF), u ^ msb)


def _make_kernel(B, L, D):
    n_vec = L // LANES

    mesh = plsc.VectorSubcoreMesh(core_axis_name="c", subcore_axis_name="s")

    @functools.partial(
        pl.kernel,
        mesh=mesh,
        compiler_params=pltpu.CompilerParams(needs_layout_passes=False),
        out_type=(
            jax.ShapeDtypeStruct((B, TOPK), jnp.int32),
            jax.ShapeDtypeStruct((B, TOPK, D), jnp.float32),
            jax.ShapeDtypeStruct((B, TOPK), jnp.float32),
        ),
        scratch_types=[
            pltpu.VMEM((L,), jnp.float32),
            pltpu.VMEM((L + LANES,), jnp.float32),
            pltpu.VMEM((L + LANES,), jnp.int32),
            pltpu.VMEM((L + LANES,), jnp.int32),
            pltpu.VMEM((LANES * HSTRIDE,), jnp.int32),
            pltpu.VMEM((LANES * HSTRIDE,), jnp.int32),
            pltpu.VMEM((LANES * HSTRIDE,), jnp.int32),
            pltpu.VMEM((LANES * HSTRIDE,), jnp.int32),
            pltpu.VMEM((L // 2,), jnp.float32),
            pltpu.VMEM((L // 2,), jnp.int32),
            pltpu.VMEM((BINS,), jnp.int32),
            pltpu.VMEM((BINS,), jnp.int32),
            pltpu.VMEM((TOPK,), jnp.int32),
            pltpu.VMEM((TOPK,), jnp.float32),
            pltpu.VMEM((TOPK, D), jnp.float32),
            pltpu.SemaphoreType.DMA,
            pltpu.SemaphoreType.DMA,
            pltpu.SemaphoreType.DMA,
        ],
    )
    def run(scores_hbm, hidden_hbm, idx_out, hid_out, scr_out,
            srow, cflt, ckey, cidx, h16, h16b, h16c, h16d, cfltB, cidxB,
            hist, rcum, selidx, selscr, hrows, sem, sem2, osem):
        nc = 2
        wid = lax.axis_index("s") * nc + lax.axis_index("c")

        @pl.when(wid < B)
        def _body():
            r = wid
            stage = pltpu.async_copy(scores_hbm.at[r], srow, sem)

            lane = lax.iota(jnp.int32, LANES)
            lane_off = lane * HSTRIDE
            ones = jnp.ones((LANES,), jnp.int32)

            def zero_body(j, _):
                z = jnp.zeros((LANES,), jnp.int32)
                sl = pl.ds(j * LANES, LANES)
                h16[sl] = z
                h16b[sl] = z
                h16c[sl] = z
                h16d[sl] = z
                return 0
            lax.fori_loop(0, (LANES * HSTRIDE) // LANES, zero_body, 0)
            stage.wait()

            banks = (h16, h16b, h16c, h16d)
            SAMPLE = 8

            def l0_body(i, _):
                for u in range(UNROLL):
                    off = (i * UNROLL + u) * SAMPLE * LANES
                    v = srow[pl.ds(off, LANES)]
                    key = _mono_u32(v)
                    digit = (key >> jnp.uint32(24)).astype(jnp.int32)
                    plsc.addupdate_scatter(banks[u % 4], [lane_off + digit],
                                           ones)
                return 0
            lax.fori_loop(0, n_vec // (SAMPLE * UNROLL), l0_body, 0)

            def reduce_hist(nbanks):
                zeros = jnp.zeros((LANES,), jnp.int32)

                def red_body(j, _):
                    acc = jnp.zeros((LANES,), jnp.int32)
                    for l in range(LANES):
                        sl = pl.ds(l * HSTRIDE + j * LANES, LANES)
                        for b in range(nbanks):
                            acc = acc + banks[b][sl]
                        h16[sl] = zeros
                    hist[pl.ds(j * LANES, LANES)] = acc
                    return 0
                lax.fori_loop(0, BINS // LANES, red_body, 0)

            def pick_digit(rem):
                carry = jnp.zeros((LANES,), jnp.int32)
                cnt = jnp.zeros((LANES,), jnp.int32)
                for j in range(BINS // LANES - 1, -1, -1):
                    h = hist[pl.ds(j * LANES, LANES)]
                    rc = lax.rev(plsc.cumsum(lax.rev(h, (0,))) + carry, (0,))
                    rcum[pl.ds(j * LANES, LANES)] = rc
                    cnt = cnt + plsc.all_reduce_population_count(rc >= rem)
                    carry = carry + jnp.sum(h)
                d = cnt - 1
                rcum_d = plsc.load_gather(rcum, [d])
                hist_d = plsc.load_gather(hist, [d])
                rem = rem - (rcum_d - hist_d)
                return d, rem

            rem = jnp.full((LANES,), TOPK, jnp.int32)
            reduce_hist(4)
            d0, _ = pick_digit(rem)
            d0_u = d0.astype(jnp.uint32)

            k0 = d0_u << jnp.uint32(24)
            msb = jnp.uint32(0x80000000)
            f_lo = lax.bitcast_convert_type(
                jnp.where(k0 >= msb, k0 ^ msb, k0 ^ jnp.uint32(0xFFFFFFFF)),
                jnp.float32)

            halfL = L // 2

            def l1_body(i, c):
                ca, cb = c
                for u in range(2):
                    offa = i * (LANES * 2) + u * LANES
                    offb = halfL + offa
                    va = srow[pl.ds(offa, LANES)]
                    vb = srow[pl.ds(offb, LANES)]
                    ma = va >= f_lo
                    mb = vb >= f_lo
                    cnta = plsc.cumsum(ma.astype(jnp.int32))
                    cntb = plsc.cumsum(mb.astype(jnp.int32))
                    plsc.store_scatter(cflt, [ca + cnta - 1], va, mask=ma)
                    plsc.store_scatter(cidx, [ca + cnta - 1], lane + offa,
                                       mask=ma)
                    plsc.store_scatter(cfltB, [cb + cntb - 1], vb, mask=mb)
                    plsc.store_scatter(cidxB, [cb + cntb - 1], lane + offb,
                                       mask=mb)
                    ca = ca + plsc.all_reduce_population_count(ma)
                    cb = cb + plsc.all_reduce_population_count(mb)
                return ca, cb
            zero_v = jnp.zeros((LANES,), jnp.int32)
            ca, cb = lax.fori_loop(0, halfL // (LANES * 2), l1_body,
                                   (zero_v, zero_v))
            n_a = jnp.max(ca)
            n_cv = ca + cb
            n_c = jnp.max(n_cv)
            ncv_vec = (n_c + LANES - 1) // LANES

            def app_body(j, _):
                src = pl.ds(j * LANES, LANES)
                dst = pl.ds(n_a + j * LANES, LANES)
                cflt[dst] = cfltB[src]
                cidx[dst] = cidxB[src]
                return 0
            lax.fori_loop(0, (jnp.max(cb) + LANES - 1) // LANES, app_body, 0)

            prefix = jnp.zeros((LANES,), jnp.uint32)
            for shift in (24, 16, 8, 0):
                hi_sh = shift + 8
                pfx = prefix

                def lv_body(i, _):
                    sl = pl.ds(i * LANES, LANES)
                    if shift == 24:
                        key = _mono_u32(cflt[sl])
                        ckey[sl] = lax.bitcast_convert_type(key, jnp.int32)
                        m = (lane + i * LANES) < n_cv
                    else:
                        key = lax.bitcast_convert_type(ckey[sl], jnp.uint32)
                        valid = (lane + i * LANES) < n_cv
                        m = jnp.logical_and(
                            valid,
                            (key >> jnp.uint32(hi_sh)) == (pfx >> jnp.uint32(hi_sh)))
                    digit = ((key >> jnp.uint32(shift)) & jnp.uint32(0xFF)).astype(jnp.int32)
                    plsc.addupdate_scatter(h16, [lane_off + digit], ones, mask=m)
                    return 0
                lax.fori_loop(0, ncv_vec, lv_body, 0)
                reduce_hist(1)
                d, rem = pick_digit(rem)
                prefix = prefix | (d.astype(jnp.uint32) << jnp.uint32(shift))

            thresh = prefix
            need_eq = rem

            def sel_body(i, c):
                cursor, eqseen = c
                key = lax.bitcast_convert_type(
                    ckey[pl.ds(i * LANES, LANES)], jnp.uint32)
                idxv = cidx[pl.ds(i * LANES, LANES)]
                valid = (lane + i * LANES) < n_cv
                m_gt = jnp.logical_and(valid, key > thresh)
                m_eq = jnp.logical_and(valid, key == thresh)
                inc_eq = plsc.cumsum(m_eq.astype(jnp.int32))
                accept = jnp.logical_and(m_eq, (eqseen + inc_eq) <= need_eq)
                take = jnp.logical_or(m_gt, accept)
                tc = plsc.cumsum(take.astype(jnp.int32))
                pos = cursor + tc - 1
                plsc.store_scatter(selidx, [pos], idxv, mask=take)
                cursor = cursor + plsc.all_reduce_population_count(take)
                eqseen = eqseen + plsc.all_reduce_population_count(m_eq)
                return cursor, eqseen
            lax.fori_loop(0, ncv_vec, sel_body, (zero_v, zero_v))

            half = TOPK // 2
            g0 = pltpu.async_copy(
                hidden_hbm.at[r].at[selidx.at[pl.ds(0, half)]],
                hrows.at[pl.ds(0, half)], sem)
            g1 = pltpu.async_copy(
                hidden_hbm.at[r].at[selidx.at[pl.ds(half, half)]],
                hrows.at[pl.ds(half, half)], sem2)

            for j in range(TOPK // LANES):
                iv = selidx[pl.ds(j * LANES, LANES)]
                selscr[pl.ds(j * LANES, LANES)] = plsc.load_gather(srow, [iv])
            pltpu.sync_copy(selidx, idx_out.at[r])
            pltpu.sync_copy(selscr, scr_out.at[r])

            g0.wait()
            o0 = pltpu.async_copy(hrows.at[pl.ds(0, half)],
                                  hid_out.at[r].at[pl.ds(0, half)], osem)
            g1.wait()
            o1 = pltpu.async_copy(hrows.at[pl.ds(half, half)],
                                  hid_out.at[r].at[pl.ds(half, half)], osem)
            o0.wait()
            o1.wait()

    return run


def kernel(scores, hidden_states):
    B, L = scores.shape
    D = hidden_states.shape[-1]
    run = _make_kernel(B, L, D)
    return run(scores, hidden_states)


# speedup vs baseline: 1.1794x; 1.1794x over previous
import functools

import jax
import jax.numpy as jnp
from jax import lax
from jax.experimental import pallas as pl
from jax.experimental.pallas import tpu as pltpu
from jax.experimental.pallas import tpu_sc as plsc

TOPK = 64
LANES = 16
BINS = 256
HSTRIDE = BINS + 17
UNROLL = 4


def _mono_u32(v):
    u = lax.bitcast_convert_type(v, jnp.uint32)
    msb = jnp.uint32(0x80000000)
    return jnp.where(u >= msb, u ^ jnp.uint32(0xFFFFFFFF), u ^ msb)


def _make_kernel(B, L, D):
    n_vec = L // LANES

    mesh = plsc.VectorSubcoreMesh(core_axis_name="c", subcore_axis_name="s")

    @functools.partial(
        pl.kernel,
        mesh=mesh,
        compiler_params=pltpu.CompilerParams(needs_layout_passes=False),
        out_type=(
            jax.ShapeDtypeStruct((B, TOPK), jnp.int32),
            jax.ShapeDtypeStruct((B, TOPK, D), jnp.float32),
            jax.ShapeDtypeStruct((B, TOPK), jnp.float32),
        ),
        scratch_types=[
            pltpu.VMEM((L,), jnp.float32),
            pltpu.VMEM((L + LANES,), jnp.float32),
            pltpu.VMEM((L + LANES,), jnp.int32),
            pltpu.VMEM((L + LANES,), jnp.int32),
            pltpu.VMEM((LANES * HSTRIDE,), jnp.int32),
            pltpu.VMEM((LANES * HSTRIDE,), jnp.int32),
            pltpu.VMEM((LANES * HSTRIDE,), jnp.int32),
            pltpu.VMEM((LANES * HSTRIDE,), jnp.int32),
            pltpu.VMEM((L // 2,), jnp.float32),
            pltpu.VMEM((L // 2,), jnp.int32),
            pltpu.VMEM((BINS,), jnp.int32),
            pltpu.VMEM((BINS,), jnp.int32),
            pltpu.VMEM((TOPK,), jnp.int32),
            pltpu.VMEM((TOPK,), jnp.float32),
            pltpu.VMEM((TOPK, D), jnp.float32),
            pltpu.SemaphoreType.DMA,
            pltpu.SemaphoreType.DMA,
            pltpu.SemaphoreType.DMA,
        ],
    )
    def run(scores_hbm, hidden_hbm, idx_out, hid_out, scr_out,
            srow, cflt, ckey, cidx, h16, h16b, h16c, h16d, cfltB, cidxB,
            hist, rcum, selidx, selscr, hrows, sem, sem2, osem):
        nc = 2
        wid = lax.axis_index("s") * nc + lax.axis_index("c")

        @pl.when(wid < B)
        def _body():
            r = wid
            stage = pltpu.async_copy(scores_hbm.at[r], srow, sem)

            lane = lax.iota(jnp.int32, LANES)
            lane_off = lane * HSTRIDE
            ones = jnp.ones((LANES,), jnp.int32)

            def zero_body(j, _):
                z = jnp.zeros((LANES,), jnp.int32)
                sl = pl.ds(j * LANES, LANES)
                h16[sl] = z
                h16b[sl] = z
                h16c[sl] = z
                h16d[sl] = z
                return 0
            lax.fori_loop(0, (LANES * HSTRIDE) // LANES, zero_body, 0)
            stage.wait()

            banks = (h16, h16b, h16c, h16d)
            SAMPLE = 2

            def l0_body(i, _):
                for u in range(UNROLL):
                    off = (i * UNROLL + u) * SAMPLE * LANES
                    v = srow[pl.ds(off, LANES)]
                    key = _mono_u32(v)
                    digit = (key >> jnp.uint32(24)).astype(jnp.int32)
                    plsc.addupdate_scatter(banks[u % 4], [lane_off + digit],
                                           ones)
                return 0
            lax.fori_loop(0, n_vec // (SAMPLE * UNROLL), l0_body, 0)

            def reduce_hist(nbanks):
                zeros = jnp.zeros((LANES,), jnp.int32)

                def red_body(j, _):
                    acc = jnp.zeros((LANES,), jnp.int32)
                    for l in range(LANES):
                        sl = pl.ds(l * HSTRIDE + j * LANES, LANES)
                        for b in range(nbanks):
                            acc = acc + banks[b][sl]
                        h16[sl] = zeros
                    hist[pl.ds(j * LANES, LANES)] = acc
                    return 0
                lax.fori_loop(0, BINS // LANES, red_body, 0)

            def pick_digit(rem):
                carry = jnp.zeros((LANES,), jnp.int32)
                cnt = jnp.zeros((LANES,), jnp.int32)
                for j in range(BINS // LANES - 1, -1, -1):
                    h = hist[pl.ds(j * LANES, LANES)]
                    rc = lax.rev(plsc.cumsum(lax.rev(h, (0,))) + carry, (0,))
                    rcum[pl.ds(j * LANES, LANES)] = rc
                    cnt = cnt + plsc.all_reduce_population_count(rc >= rem)
                    carry = carry + jnp.sum(h)
                d = cnt - 1
                rcum_d = plsc.load_gather(rcum, [d])
                hist_d = plsc.load_gather(hist, [d])
                rem = rem - (rcum_d - hist_d)
                return d, rem

            rem = jnp.full((LANES,), TOPK, jnp.int32)
            reduce_hist(4)
            d0, _ = pick_digit(rem)
            d0_u = d0.astype(jnp.uint32)

            k0 = d0_u << jnp.uint32(24)
            msb = jnp.uint32(0x80000000)
            f_lo = lax.bitcast_convert_type(
                jnp.where(k0 >= msb, k0 ^ msb, k0 ^ jnp.uint32(0xFFFFFFFF)),
                jnp.float32)

            halfL = L // 2

            def l1_body(i, c):
                ca, cb = c
                for u in range(2):
                    offa = i * (LANES * 2) + u * LANES
                    offb = halfL + offa
                    va = srow[pl.ds(offa, LANES)]
                    vb = srow[pl.ds(offb, LANES)]
                    ma = va >= f_lo
                    mb = vb >= f_lo
                    cnta = plsc.cumsum(ma.astype(jnp.int32))
                    cntb = plsc.cumsum(mb.astype(jnp.int32))
                    plsc.store_scatter(cflt, [ca + cnta - 1], va, mask=ma)
                    plsc.store_scatter(cidx, [ca + cnta - 1], lane + offa,
                                       mask=ma)
                    plsc.store_scatter(cfltB, [cb + cntb - 1], vb, mask=mb)
                    plsc.store_scatter(cidxB, [cb + cntb - 1], lane + offb,
                                       mask=mb)
                    ca = ca + plsc.all_reduce_population_count(ma)
                    cb = cb + plsc.all_reduce_population_count(mb)
                return ca, cb
            zero_v = jnp.zeros((LANES,), jnp.int32)
            ca, cb = lax.fori_loop(0, halfL // (LANES * 2), l1_body,
                                   (zero_v, zero_v))
            n_a = jnp.max(ca)
            n_cv = ca + cb
            n_c = jnp.max(n_cv)
            ncv_vec = (n_c + LANES - 1) // LANES

            def app_body(j, _):
                src = pl.ds(j * LANES, LANES)
                dst = pl.ds(n_a + j * LANES, LANES)
                cflt[dst] = cfltB[src]
                cidx[dst] = cidxB[src]
                return 0
            lax.fori_loop(0, (jnp.max(cb) + LANES - 1) // LANES, app_body, 0)

            prefix = jnp.zeros((LANES,), jnp.uint32)
            for shift in (24, 16, 8, 0):
                hi_sh = shift + 8
                pfx = prefix

                def lv_body(i, _):
                    sl = pl.ds(i * LANES, LANES)
                    if shift == 24:
                        key = _mono_u32(cflt[sl])
                        ckey[sl] = lax.bitcast_convert_type(key, jnp.int32)
                        m = (lane + i * LANES) < n_cv
                    else:
                        key = lax.bitcast_convert_type(ckey[sl], jnp.uint32)
                        valid = (lane + i * LANES) < n_cv
                        m = jnp.logical_and(
                            valid,
                            (key >> jnp.uint32(hi_sh)) == (pfx >> jnp.uint32(hi_sh)))
                    digit = ((key >> jnp.uint32(shift)) & jnp.uint32(0xFF)).astype(jnp.int32)
                    plsc.addupdate_scatter(h16, [lane_off + digit], ones, mask=m)
                    return 0
                lax.fori_loop(0, ncv_vec, lv_body, 0)
                reduce_hist(1)
                d, rem = pick_digit(rem)
                prefix = prefix | (d.astype(jnp.uint32) << jnp.uint32(shift))

            thresh = prefix
            need_eq = rem

            def sel_body(i, c):
                cursor, eqseen = c
                key = lax.bitcast_convert_type(
                    ckey[pl.ds(i * LANES, LANES)], jnp.uint32)
                idxv = cidx[pl.ds(i * LANES, LANES)]
                valid = (lane + i * LANES) < n_cv
                m_gt = jnp.logical_and(valid, key > thresh)
                m_eq = jnp.logical_and(valid, key == thresh)
                inc_eq = plsc.cumsum(m_eq.astype(jnp.int32))
                accept = jnp.logical_and(m_eq, (eqseen + inc_eq) <= need_eq)
                take = jnp.logical_or(m_gt, accept)
                tc = plsc.cumsum(take.astype(jnp.int32))
                pos = cursor + tc - 1
                plsc.store_scatter(selidx, [pos], idxv, mask=take)
                cursor = cursor + plsc.all_reduce_population_count(take)
                eqseen = eqseen + plsc.all_reduce_population_count(m_eq)
                return cursor, eqseen
            lax.fori_loop(0, ncv_vec, sel_body, (zero_v, zero_v))

            half = TOPK // 2
            g0 = pltpu.async_copy(
                hidden_hbm.at[r].at[selidx.at[pl.ds(0, half)]],
                hrows.at[pl.ds(0, half)], sem)
            g1 = pltpu.async_copy(
                hidden_hbm.at[r].at[selidx.at[pl.ds(half, half)]],
                hrows.at[pl.ds(half, half)], sem2)

            for j in range(TOPK // LANES):
                iv = selidx[pl.ds(j * LANES, LANES)]
                selscr[pl.ds(j * LANES, LANES)] = plsc.load_gather(srow, [iv])
            pltpu.sync_copy(selidx, idx_out.at[r])
            pltpu.sync_copy(selscr, scr_out.at[r])

            g0.wait()
            o0 = pltpu.async_copy(hrows.at[pl.ds(0, half)],
                                  hid_out.at[r].at[pl.ds(0, half)], osem)
            g1.wait()
            o1 = pltpu.async_copy(hrows.at[pl.ds(half, half)],
                                  hid_out.at[r].at[pl.ds(half, half)], osem)
            o0.wait()
            o1.wait()

    return run


def kernel(scores, hidden_states):
    B, L = scores.shape
    D = hidden_states.shape[-1]
    run = _make_kernel(B, L, D)
    return run(scores, hidden_states)
